# Initial kernel scaffold; baseline (speedup 1.0000x reference)
import jax
import jax.numpy as jnp
from jax.experimental import pallas as pl


def kernel(gt_score, desc, seg_mask, seg, seg_confidence):
    raise NotImplementedError("write your pallas kernel here")


# speedup vs baseline: 10.4851x; 10.4851x over previous
import jax
import jax.numpy as jnp
from jax import lax
from jax.experimental import pallas as pl
from jax.experimental.pallas import tpu as pltpu

_K = 2000
_NSEG = 20
_MAXBITS = 0x3F800000
_NSTEP = 16
_CH = 256 // _NSTEP


def _body(sc_ref, mk_ref, seg_ref, desc_ref, out_ref,
          w_ref, u_ref, cnt_ref, asum_ref):
    i = pl.program_id(0)

    @pl.when(i == 0)
    def _select_stage():
        sc_all = sc_ref[...]
        mk_all = mk_ref[...]
        seg_all = seg_ref[...]
        bits_all = lax.bitcast_convert_type(sc_all, jnp.int32)

        def bs_body(_, carry):
            lo, hi = carry
            mid = (lo + hi) // 2
            cnt = jnp.sum((bits_all >= mid).astype(jnp.float32))
            ge = cnt >= float(_K)
            return (jnp.where(ge, mid, lo), jnp.where(ge, hi, mid))

        th_bits, _ = lax.fori_loop(
            0, 31, bs_body, (jnp.int32(0), jnp.int32(_MAXBITS))
        )

        ri = lax.broadcasted_iota(jnp.int32, (512, 512), 0)
        ci = lax.broadcasted_iota(jnp.int32, (512, 512), 1)
        ut = (ri <= ci).astype(jnp.float32)
        sl = (ci < ri).astype(jnp.float32)
        r2 = lax.broadcasted_iota(jnp.int32, (256, 512), 0)
        c2 = lax.broadcasted_iota(jnp.int32, (256, 512), 1)
        pool_l = (c2 // 2 == r2).astype(jnp.float32)
        r3 = lax.broadcasted_iota(jnp.int32, (512, 256), 0)
        c3 = lax.broadcasted_iota(jnp.int32, (512, 256), 1)
        pool_r = (r3 // 2 == c3).astype(jnp.float32)

        for n in range(2):
            bits = bits_all[n]
            sc = sc_all[n]
            sg = seg_all[n]
            flag = jnp.where(bits >= th_bits, 1.0, 0.0) * mk_all[n]
            csum = jnp.dot(flag, ut, preferred_element_type=jnp.float32)
            rowsum = csum[:, 511:512]
            roff = jnp.dot(sl, rowsum, preferred_element_type=jnp.float32)
            pref = (csum - flag) + roff
            taken = flag * (pref < float(_K)).astype(jnp.float32)
            s = jnp.clip(jnp.clip(sc, 0.0005, 1.0) * 2.0 + 0.5, 0.0005, 1.0)
            tsc = taken * s
            for g in range(_NSEG):
                eg = (sg == g).astype(jnp.float32)
                cnt_ref[n, g] = jnp.sum(taken * eg)
                tg = tsc * eg
                asum_ref[n, g] = jnp.sum(tg)
                w_ref[n, g] = jnp.dot(
                    jnp.dot(pool_l, tg, preferred_element_type=jnp.float32),
                    pool_r,
                    preferred_element_type=jnp.float32,
                )

    for n in range(2):
        wsl = w_ref[n, :, pl.ds(i * _CH, _CH), :]
        wmat = wsl.reshape(_NSEG, _CH * 256)
        dmat = desc_ref[n].reshape(128, _CH * 256)
        part = lax.dot_general(
            wmat, dmat, (((1,), (1,)), ((), ())),
            preferred_element_type=jnp.float32,
        )

        @pl.when(i == 0)
        def _init():
            u_ref[n] = part

        @pl.when(i > 0)
        def _acc():
            u_ref[n] = u_ref[n] + part

    @pl.when(i == _NSTEP - 1)
    def _finish():
        n1 = [cnt_ref[0, g] for g in range(_NSEG)]
        n2 = [cnt_ref[1, g] for g in range(_NSEG)]
        a1 = [asum_ref[0, g] for g in range(_NSEG)]
        a2 = [asum_ref[1, g] for g in range(_NSEG)]
        n_pos = sum(x * y for x, y in zip(n1, n2))
        ntot1 = sum(n1)
        ntot2 = sum(n2)
        sa = sum(x * y for x, y in zip(a1, a2))
        ta = sum(a1) * sum(a2)
        gmat = lax.dot_general(
            u_ref[0], u_ref[1], (((1,), (1,)), ((), ())),
            preferred_element_type=jnp.float32,
        )
        gr = lax.broadcasted_iota(jnp.int32, (_NSEG, _NSEG), 0)
        gc = lax.broadcasted_iota(jnp.int32, (_NSEG, _NSEG), 1)
        eye = (gr == gc).astype(jnp.float32)
        s_u = jnp.sum(gmat * eye)
        t_u = jnp.sum(gmat)
        s_pos = 2.0 * sa - 2.0 * s_u
        s_all = 2.0 * ta - 2.0 * t_u
        s_neg = s_all - s_pos
        n_neg = ntot1 * ntot2 - n_pos
        pos_dist = jnp.where(n_pos > 0.0, s_pos / jnp.maximum(n_pos, 1.0), 0.0)
        neg_dist = jnp.where(n_neg > 0.0, s_neg / jnp.maximum(n_neg, 1.0), 0.0)
        out_ref[0, 0] = 1.0 + pos_dist - neg_dist


def kernel(gt_score, desc, seg_mask, seg, seg_confidence):
    del seg_confidence
    mk = seg_mask.astype(jnp.float32)
    sg = seg.astype(jnp.int32)
    out = pl.pallas_call(
        _body,
        grid=(_NSTEP,),
        in_specs=[
            pl.BlockSpec((2, 512, 512), lambda i: (0, 0, 0)),
            pl.BlockSpec((2, 512, 512), lambda i: (0, 0, 0)),
            pl.BlockSpec((2, 512, 512), lambda i: (0, 0, 0)),
            pl.BlockSpec((2, 128, _CH, 256), lambda i: (0, 0, i, 0)),
        ],
        out_shape=jax.ShapeDtypeStruct((1, 1), jnp.float32),
        out_specs=pl.BlockSpec((1, 1), lambda i: (0, 0),
                               memory_space=pltpu.SMEM),
        scratch_shapes=[
            pltpu.VMEM((2, _NSEG, 256, 256), jnp.float32),
            pltpu.VMEM((2, _NSEG, 128), jnp.float32),
            pltpu.SMEM((2, _NSEG), jnp.float32),
            pltpu.SMEM((2, _NSEG), jnp.float32),
        ],
        compiler_params=pltpu.CompilerParams(
            vmem_limit_bytes=60 * 1024 * 1024,
        ),
    )(gt_score, mk, sg, desc)
    return out[0, 0]
